# Initial kernel scaffold; baseline (speedup 1.0000x reference)
import jax
import jax.numpy as jnp
from jax.experimental import pallas as pl


def kernel(h, nodeset, nb_nodes, nb_weights, Q_w, Q_b, W_w, W_b):
    raise NotImplementedError("write your pallas kernel here")


# speedup vs baseline: 1.9292x; 1.9292x over previous
import functools

import jax
import jax.numpy as jnp
from jax import lax
from jax.experimental import pallas as pl
from jax.experimental.pallas import tpu as pltpu
from jax.experimental.pallas import tpu_sc as plsc


def _leaky(x):
    return jnp.where(x >= 0, x, 0.01 * x)


def _hq_body(h_ref, qw_ref, qb_ref, out_ref):
    x = lax.dot_general(h_ref[...], qw_ref[...],
                        (((1,), (1,)), ((), ())),
                        preferred_element_type=jnp.float32)
    out_ref[...] = _leaky(x + qb_ref[...])


def _make_hq(h, Q_w, Q_b):
    n, in_f = h.shape
    hid = Q_w.shape[0]
    blk = 2000
    return pl.pallas_call(
        _hq_body,
        grid=(n // blk,),
        in_specs=[
            pl.BlockSpec((blk, in_f), lambda i: (i, 0)),
            pl.BlockSpec((hid, in_f), lambda i: (0, 0)),
            pl.BlockSpec((1, hid), lambda i: (0, 0)),
        ],
        out_specs=pl.BlockSpec((blk, hid), lambda i: (i, 0)),
        out_shape=jax.ShapeDtypeStruct((n, hid), jnp.float32),
    )(h, Q_w, Q_b.reshape(1, hid))


def _make_sc_agg(in_f, hid, bp, t, nw, nc):
    L = 16
    b_per_w = bp // nw
    C = 8
    C2 = 64
    n_j = hid // L

    mesh = plsc.VectorSubcoreMesh(core_axis_name="c", subcore_axis_name="s")

    @functools.partial(
        pl.kernel,
        mesh=mesh,
        compiler_params=pltpu.CompilerParams(use_tc_tiling_on_sc=False),
        out_type=[
            jax.ShapeDtypeStruct((bp, hid), jnp.float32),
            jax.ShapeDtypeStruct((bp, in_f), jnp.float32),
        ],
        scratch_types=[
            pltpu.VMEM((C * t,), jnp.int32),
            pltpu.VMEM((C * t, hid), jnp.float32),
            pltpu.VMEM((C * t,), jnp.float32),
            pltpu.VMEM((C, hid), jnp.float32),
            pltpu.VMEM((C2,), jnp.int32),
            pltpu.VMEM((C2, in_f), jnp.float32),
            pltpu.SemaphoreType.DMA,
        ],
    )
    def sc_agg(hq_hbm, h_hbm, ns_hbm, nb_hbm, w_hbm, agg_out, hns_out,
               nbidx_v, rows_v, w_v, agg_v, nsidx_v, hrows_v, sem):
        wid = lax.axis_index("s") * nc + lax.axis_index("c")
        base = wid * b_per_w

        def nb_chunk(c, _):
            nbase = base + c * C
            pltpu.sync_copy(nb_hbm.at[pl.ds(nbase * t, C * t)], nbidx_v)
            pltpu.sync_copy(w_hbm.at[pl.ds(nbase * t, C * t)], w_v)
            cp0 = pltpu.async_copy(hq_hbm.at[nbidx_v.at[pl.ds(0, 128)]],
                                   rows_v.at[pl.ds(0, 128)], sem)
            cp1 = pltpu.async_copy(hq_hbm.at[nbidx_v.at[pl.ds(128, 128)]],
                                   rows_v.at[pl.ds(128, 128)], sem)
            cp0.wait()
            cp1.wait()

            def node(i, _):
                accs = [jnp.zeros((L,), jnp.float32) for _ in range(n_j)]
                wrow = [w_v[pl.ds(i * t + g * L, L)] for g in range(t // L)]
                for tt in range(t):
                    wspl = jnp.full((L,), wrow[tt // L][tt % L],
                                    dtype=jnp.float32)
                    for j in range(n_j):
                        accs[j] = accs[j] + wspl * rows_v[i * t + tt,
                                                          pl.ds(j * L, L)]
                for j in range(n_j):
                    agg_v[i, pl.ds(j * L, L)] = accs[j]
                return 0

            lax.fori_loop(0, C, node, 0)
            pltpu.sync_copy(agg_v, agg_out.at[pl.ds(nbase, C)])
            return 0

        lax.fori_loop(0, b_per_w // C, nb_chunk, 0)

        def ns_chunk(c, _):
            nbase = base + c * C2
            pltpu.sync_copy(ns_hbm.at[pl.ds(nbase, C2)], nsidx_v)
            pltpu.async_copy(h_hbm.at[nsidx_v], hrows_v, sem).wait()
            pltpu.sync_copy(hrows_v, hns_out.at[pl.ds(nbase, C2)])
            return 0

        lax.fori_loop(0, b_per_w // C2, ns_chunk, 0)

    return sc_agg


def _out_body(hns_ref, agg_ref, nbw_ref, w_ref, wb_ref, out_ref, *, in_f):
    ws = jnp.sum(nbw_ref[...], axis=1, keepdims=True)
    ws = jnp.where(ws == 0, 1.0, ws)
    agg = agg_ref[...] / ws
    w = w_ref[...]
    y = lax.dot_general(hns_ref[...], w[:, :in_f],
                        (((1,), (1,)), ((), ())),
                        preferred_element_type=jnp.float32)
    y = y + lax.dot_general(agg, w[:, in_f:],
                            (((1,), (1,)), ((), ())),
                            preferred_element_type=jnp.float32)
    y = _leaky(y + wb_ref[...])
    norm = jnp.sqrt(jnp.sum(y * y, axis=1, keepdims=True))
    norm = jnp.where(norm == 0, 1.0, norm)
    out_ref[...] = y / norm


def kernel(h, nodeset, nb_nodes, nb_weights, Q_w, Q_b, W_w, W_b):
    n_total, in_f = h.shape
    b, t = nb_nodes.shape
    hid = Q_w.shape[0]
    out_f = W_w.shape[0]

    info = plsc.get_sparse_core_info()
    nc, n_sub = info.num_cores, info.num_subcores
    nw = nc * n_sub

    chunk = 64 * nw
    bp = ((b + chunk - 1) // chunk) * chunk
    pad = bp - b
    ns_p = jnp.pad(nodeset.astype(jnp.int32), (0, pad))
    nb_p = jnp.pad(nb_nodes.astype(jnp.int32).reshape(-1), (0, pad * t))
    w_p = jnp.pad(nb_weights.reshape(-1), (0, pad * t))

    hq = _make_hq(h, Q_w, Q_b)
    agg, hns = _make_sc_agg(in_f, hid, bp, t, nw, nc)(
        hq, h, ns_p, nb_p, w_p)

    blk = 1000
    out = pl.pallas_call(
        functools.partial(_out_body, in_f=in_f),
        grid=(b // blk,),
        in_specs=[
            pl.BlockSpec((blk, in_f), lambda i: (i, 0)),
            pl.BlockSpec((blk, hid), lambda i: (i, 0)),
            pl.BlockSpec((blk, t), lambda i: (i, 0)),
            pl.BlockSpec((out_f, in_f + hid), lambda i: (0, 0)),
            pl.BlockSpec((1, out_f), lambda i: (0, 0)),
        ],
        out_specs=pl.BlockSpec((blk, out_f), lambda i: (i, 0)),
        out_shape=jax.ShapeDtypeStruct((b, out_f), jnp.float32),
    )(hns, agg, nb_weights, W_w, W_b.reshape(1, out_f))
    return out


# speedup vs baseline: 2.2364x; 1.1592x over previous
import functools

import jax
import jax.numpy as jnp
from jax import lax
from jax.experimental import pallas as pl
from jax.experimental.pallas import tpu as pltpu
from jax.experimental.pallas import tpu_sc as plsc


def _leaky(x):
    return jnp.where(x >= 0, x, 0.01 * x)


def _hq_body(h_ref, qw_ref, qb_ref, out_ref):
    x = lax.dot_general(h_ref[...], qw_ref[...],
                        (((1,), (1,)), ((), ())),
                        preferred_element_type=jnp.float32)
    out_ref[...] = _leaky(x + qb_ref[...])


def _make_hq(h, Q_w, Q_b):
    n, in_f = h.shape
    hid = Q_w.shape[0]
    blk = 2000
    return pl.pallas_call(
        _hq_body,
        grid=(n // blk,),
        in_specs=[
            pl.BlockSpec((blk, in_f), lambda i: (i, 0)),
            pl.BlockSpec((hid, in_f), lambda i: (0, 0)),
            pl.BlockSpec((1, hid), lambda i: (0, 0)),
        ],
        out_specs=pl.BlockSpec((blk, hid), lambda i: (i, 0)),
        out_shape=jax.ShapeDtypeStruct((n, hid), jnp.float32),
    )(h, Q_w, Q_b.reshape(1, hid))


def _make_sc_agg(in_f, hid, bp, t, nw, nc):
    L = 16
    b_per_w = bp // nw
    C = 16
    G = (C * t) // 128
    C2 = 64
    n_j = hid // L
    nch = b_per_w // C

    mesh = plsc.VectorSubcoreMesh(core_axis_name="c", subcore_axis_name="s")

    @functools.partial(
        pl.kernel,
        mesh=mesh,
        compiler_params=pltpu.CompilerParams(use_tc_tiling_on_sc=False),
        out_type=[
            jax.ShapeDtypeStruct((bp, hid), jnp.float32),
            jax.ShapeDtypeStruct((bp, in_f), jnp.float32),
        ],
        scratch_types=[
            pltpu.VMEM((b_per_w * t,), jnp.int32),
            pltpu.VMEM((b_per_w * t,), jnp.float32),
            pltpu.VMEM((C * t, hid), jnp.float32),
            pltpu.VMEM((C * t, hid), jnp.float32),
            pltpu.VMEM((C, hid), jnp.float32),
            pltpu.VMEM((C, hid), jnp.float32),
            pltpu.VMEM((C2,), jnp.int32),
            pltpu.VMEM((C2, in_f), jnp.float32),
            pltpu.SemaphoreType.DMA,
            pltpu.SemaphoreType.DMA,
            pltpu.SemaphoreType.DMA,
            pltpu.SemaphoreType.DMA,
            pltpu.SemaphoreType.DMA,
        ],
    )
    def sc_agg(hq_hbm, h_hbm, ns_hbm, nb_hbm, w_hbm, agg_out, hns_out,
               idx_all, w_all, rows_a, rows_b, agg_a, agg_b,
               nsidx_v, hrows_v, sem_a, sem_b, sem_oa, sem_ob, sem_ns):
        wid = lax.axis_index("s") * nc + lax.axis_index("c")
        base = wid * b_per_w

        pltpu.sync_copy(nb_hbm.at[pl.ds(base * t, b_per_w * t)], idx_all)
        pltpu.sync_copy(w_hbm.at[pl.ds(base * t, b_per_w * t)], w_all)

        def start_gathers(c, rows_v, sem):
            for k in range(G):
                pltpu.async_copy(
                    hq_hbm.at[idx_all.at[pl.ds(c * C * t + k * 128, 128)]],
                    rows_v.at[pl.ds(k * 128, 128)], sem)

        def wait_gathers(rows_v, sem):
            for k in range(G):
                pltpu.make_async_copy(
                    hq_hbm.at[idx_all.at[pl.ds(k * 128, 128)]],
                    rows_v.at[pl.ds(k * 128, 128)], sem).wait()

        def compute(c, rows_v, agg_v):
            def node(i, _):
                nb_off = (c * C + i) * t
                accs = [jnp.zeros((L,), jnp.float32) for _ in range(n_j)]
                wrow = [w_all[pl.ds(nb_off + g * L, L)]
                        for g in range(t // L)]
                for tt in range(t):
                    wspl = jnp.full((L,), wrow[tt // L][tt % L],
                                    dtype=jnp.float32)
                    for j in range(n_j):
                        accs[j] = accs[j] + wspl * rows_v[i * t + tt,
                                                          pl.ds(j * L, L)]
                for j in range(n_j):
                    agg_v[i, pl.ds(j * L, L)] = accs[j]
                return 0

            lax.fori_loop(0, C, node, 0)

        def drain_out(agg_v, sem):
            pltpu.make_async_copy(agg_v, agg_out.at[pl.ds(0, C)], sem).wait()

        start_gathers(0, rows_a, sem_a)

        def pair(cp, _):
            c0 = 2 * cp
            c1 = c0 + 1
            start_gathers(c1, rows_b, sem_b)

            @pl.when(cp > 0)
            def _():
                drain_out(agg_a, sem_oa)
            wait_gathers(rows_a, sem_a)
            compute(c0, rows_a, agg_a)
            pltpu.async_copy(agg_a, agg_out.at[pl.ds(base + c0 * C, C)],
                             sem_oa)

            @pl.when(c1 + 1 < nch)
            def _():
                start_gathers(c1 + 1, rows_a, sem_a)

            @pl.when(cp > 0)
            def _():
                drain_out(agg_b, sem_ob)
            wait_gathers(rows_b, sem_b)
            compute(c1, rows_b, agg_b)
            pltpu.async_copy(agg_b, agg_out.at[pl.ds(base + c1 * C, C)],
                             sem_ob)
            return 0

        lax.fori_loop(0, nch // 2, pair, 0)
        drain_out(agg_a, sem_oa)
        drain_out(agg_b, sem_ob)

        def ns_chunk(c, _):
            nbase = base + c * C2
            pltpu.sync_copy(ns_hbm.at[pl.ds(nbase, C2)], nsidx_v)
            pltpu.async_copy(h_hbm.at[nsidx_v], hrows_v, sem_ns).wait()
            pltpu.sync_copy(hrows_v, hns_out.at[pl.ds(nbase, C2)])
            return 0

        lax.fori_loop(0, b_per_w // C2, ns_chunk, 0)

    return sc_agg


def _out_body(hns_ref, agg_ref, nbw_ref, w_ref, wb_ref, out_ref, *, in_f):
    ws = jnp.sum(nbw_ref[...], axis=1, keepdims=True)
    ws = jnp.where(ws == 0, 1.0, ws)
    agg = agg_ref[...] / ws
    w = w_ref[...]
    y = lax.dot_general(hns_ref[...], w[:, :in_f],
                        (((1,), (1,)), ((), ())),
                        preferred_element_type=jnp.float32)
    y = y + lax.dot_general(agg, w[:, in_f:],
                            (((1,), (1,)), ((), ())),
                            preferred_element_type=jnp.float32)
    y = _leaky(y + wb_ref[...])
    norm = jnp.sqrt(jnp.sum(y * y, axis=1, keepdims=True))
    norm = jnp.where(norm == 0, 1.0, norm)
    out_ref[...] = y / norm


def kernel(h, nodeset, nb_nodes, nb_weights, Q_w, Q_b, W_w, W_b):
    n_total, in_f = h.shape
    b, t = nb_nodes.shape
    hid = Q_w.shape[0]
    out_f = W_w.shape[0]

    info = plsc.get_sparse_core_info()
    nc, n_sub = info.num_cores, info.num_subcores
    nw = nc * n_sub

    chunk = 64 * nw
    bp = ((b + chunk - 1) // chunk) * chunk
    pad = bp - b
    ns_p = jnp.pad(nodeset.astype(jnp.int32), (0, pad))
    nb_p = jnp.pad(nb_nodes.astype(jnp.int32).reshape(-1), (0, pad * t))
    w_p = jnp.pad(nb_weights.reshape(-1), (0, pad * t))

    hq = _make_hq(h, Q_w, Q_b)
    agg, hns = _make_sc_agg(in_f, hid, bp, t, nw, nc)(
        hq, h, ns_p, nb_p, w_p)

    blk = 1000
    out = pl.pallas_call(
        functools.partial(_out_body, in_f=in_f),
        grid=(b // blk,),
        in_specs=[
            pl.BlockSpec((blk, in_f), lambda i: (i, 0)),
            pl.BlockSpec((blk, hid), lambda i: (i, 0)),
            pl.BlockSpec((blk, t), lambda i: (i, 0)),
            pl.BlockSpec((out_f, in_f + hid), lambda i: (0, 0)),
            pl.BlockSpec((1, out_f), lambda i: (0, 0)),
        ],
        out_specs=pl.BlockSpec((blk, out_f), lambda i: (i, 0)),
        out_shape=jax.ShapeDtypeStruct((b, out_f), jnp.float32),
    )(hns, agg, nb_weights, W_w, W_b.reshape(1, out_f))
    return out


# speedup vs baseline: 2.2888x; 1.0234x over previous
import functools

import jax
import jax.numpy as jnp
from jax import lax
from jax.experimental import pallas as pl
from jax.experimental.pallas import tpu as pltpu
from jax.experimental.pallas import tpu_sc as plsc


def _leaky(x):
    return jnp.where(x >= 0, x, 0.01 * x)


def _hq_body(h_ref, qw_ref, qb_ref, out_ref):
    x = lax.dot_general(h_ref[...], qw_ref[...],
                        (((1,), (1,)), ((), ())),
                        preferred_element_type=jnp.float32)
    out_ref[...] = _leaky(x + qb_ref[...])


def _make_hq(h, Q_w, Q_b):
    n, in_f = h.shape
    hid = Q_w.shape[0]
    blk = 2000
    return pl.pallas_call(
        _hq_body,
        grid=(n // blk,),
        in_specs=[
            pl.BlockSpec((blk, in_f), lambda i: (i, 0)),
            pl.BlockSpec((hid, in_f), lambda i: (0, 0)),
            pl.BlockSpec((1, hid), lambda i: (0, 0)),
        ],
        out_specs=pl.BlockSpec((blk, hid), lambda i: (i, 0)),
        out_shape=jax.ShapeDtypeStruct((n, hid), jnp.float32),
    )(h, Q_w, Q_b.reshape(1, hid))


def _make_sc_ns(in_f, bp, nw, nc):
    b_per_w = bp // nw
    mesh = plsc.VectorSubcoreMesh(core_axis_name="c", subcore_axis_name="s")
    n_g = (b_per_w + 127) // 128

    @functools.partial(
        pl.kernel,
        mesh=mesh,
        out_type=jax.ShapeDtypeStruct((bp, in_f), jnp.float32),
        scratch_types=[
            pltpu.VMEM((b_per_w,), jnp.int32),
            pltpu.VMEM((b_per_w, in_f), jnp.float32),
            pltpu.SemaphoreType.DMA,
        ],
    )
    def sc_ns(h_hbm, ns_hbm, hns_out, nsidx_v, hrows_v, sem):
        wid = lax.axis_index("s") * nc + lax.axis_index("c")
        base = wid * b_per_w
        pltpu.sync_copy(ns_hbm.at[pl.ds(base, b_per_w)], nsidx_v)
        cps = []
        for k in range(n_g):
            sz = min(128, b_per_w - k * 128)
            cps.append(pltpu.async_copy(
                h_hbm.at[nsidx_v.at[pl.ds(k * 128, sz)]],
                hrows_v.at[pl.ds(k * 128, sz)], sem))
        for cp in cps:
            cp.wait()
        pltpu.sync_copy(hrows_v, hns_out.at[pl.ds(base, b_per_w)])

    return sc_ns


def _make_sc_agg(in_f, hid, bp, t, nw, nc):
    L = 16
    b_per_w = bp // nw
    C = 16
    G = (C * t) // 128
    n_j = hid // L
    nch = b_per_w // C

    mesh = plsc.VectorSubcoreMesh(core_axis_name="c", subcore_axis_name="s")

    @functools.partial(
        pl.kernel,
        mesh=mesh,
        compiler_params=pltpu.CompilerParams(use_tc_tiling_on_sc=False),
        out_type=jax.ShapeDtypeStruct((bp, hid), jnp.float32),
        scratch_types=[
            pltpu.VMEM((b_per_w * t,), jnp.int32),
            pltpu.VMEM((b_per_w * t,), jnp.float32),
            pltpu.VMEM((C * t, hid), jnp.float32),
            pltpu.VMEM((C * t, hid), jnp.float32),
            pltpu.VMEM((C, hid), jnp.float32),
            pltpu.VMEM((C, hid), jnp.float32),
            pltpu.SemaphoreType.DMA,
            pltpu.SemaphoreType.DMA,
            pltpu.SemaphoreType.DMA,
            pltpu.SemaphoreType.DMA,
        ],
    )
    def sc_agg(hq_hbm, nb_hbm, w_hbm, agg_out,
               idx_all, w_all, rows_a, rows_b, agg_a, agg_b,
               sem_a, sem_b, sem_oa, sem_ob):
        wid = lax.axis_index("s") * nc + lax.axis_index("c")
        base = wid * b_per_w

        pltpu.sync_copy(nb_hbm.at[pl.ds(base * t, b_per_w * t)], idx_all)
        pltpu.sync_copy(w_hbm.at[pl.ds(base * t, b_per_w * t)], w_all)

        def start_gathers(c, rows_v, sem):
            for k in range(G):
                pltpu.async_copy(
                    hq_hbm.at[idx_all.at[pl.ds(c * C * t + k * 128, 128)]],
                    rows_v.at[pl.ds(k * 128, 128)], sem)

        def wait_gathers(rows_v, sem):
            for k in range(G):
                pltpu.make_async_copy(
                    hq_hbm.at[idx_all.at[pl.ds(k * 128, 128)]],
                    rows_v.at[pl.ds(k * 128, 128)], sem).wait()

        def compute(c, rows_v, agg_v):
            def node(i, _):
                nb_off = (c * C + i) * t
                accs = [jnp.zeros((L,), jnp.float32) for _ in range(n_j)]
                wrow = [w_all[pl.ds(nb_off + g * L, L)]
                        for g in range(t // L)]
                for tt in range(t):
                    wspl = jnp.full((L,), wrow[tt // L][tt % L],
                                    dtype=jnp.float32)
                    for j in range(n_j):
                        accs[j] = accs[j] + wspl * rows_v[i * t + tt,
                                                          pl.ds(j * L, L)]
                for j in range(n_j):
                    agg_v[i, pl.ds(j * L, L)] = accs[j]
                return 0

            lax.fori_loop(0, C, node, 0)

        def drain_out(agg_v, sem):
            pltpu.make_async_copy(agg_v, agg_out.at[pl.ds(0, C)], sem).wait()

        start_gathers(0, rows_a, sem_a)

        def pair(cp, _):
            c0 = 2 * cp
            c1 = c0 + 1
            start_gathers(c1, rows_b, sem_b)

            @pl.when(cp > 0)
            def _():
                drain_out(agg_a, sem_oa)
            wait_gathers(rows_a, sem_a)
            compute(c0, rows_a, agg_a)
            pltpu.async_copy(agg_a, agg_out.at[pl.ds(base + c0 * C, C)],
                             sem_oa)

            @pl.when(c1 + 1 < nch)
            def _():
                start_gathers(c1 + 1, rows_a, sem_a)

            @pl.when(cp > 0)
            def _():
                drain_out(agg_b, sem_ob)
            wait_gathers(rows_b, sem_b)
            compute(c1, rows_b, agg_b)
            pltpu.async_copy(agg_b, agg_out.at[pl.ds(base + c1 * C, C)],
                             sem_ob)
            return 0

        lax.fori_loop(0, nch // 2, pair, 0)
        drain_out(agg_a, sem_oa)
        drain_out(agg_b, sem_ob)

    return sc_agg


def _out_body(hns_ref, agg_ref, nbw_ref, w_ref, wb_ref, out_ref, *, in_f):
    ws = jnp.sum(nbw_ref[...], axis=1, keepdims=True)
    ws = jnp.where(ws == 0, 1.0, ws)
    agg = agg_ref[...] / ws
    w = w_ref[...]
    y = lax.dot_general(hns_ref[...], w[:, :in_f],
                        (((1,), (1,)), ((), ())),
                        preferred_element_type=jnp.float32)
    y = y + lax.dot_general(agg, w[:, in_f:],
                            (((1,), (1,)), ((), ())),
                            preferred_element_type=jnp.float32)
    y = _leaky(y + wb_ref[...])
    norm = jnp.sqrt(jnp.sum(y * y, axis=1, keepdims=True))
    norm = jnp.where(norm == 0, 1.0, norm)
    out_ref[...] = y / norm


def kernel(h, nodeset, nb_nodes, nb_weights, Q_w, Q_b, W_w, W_b):
    n_total, in_f = h.shape
    b, t = nb_nodes.shape
    hid = Q_w.shape[0]
    out_f = W_w.shape[0]

    info = plsc.get_sparse_core_info()
    nc, n_sub = info.num_cores, info.num_subcores
    nw = nc * n_sub

    chunk = 64 * nw
    bp = ((b + chunk - 1) // chunk) * chunk
    pad = bp - b
    ns_p = jnp.pad(nodeset.astype(jnp.int32), (0, pad))
    nb_p = jnp.pad(nb_nodes.astype(jnp.int32).reshape(-1), (0, pad * t))
    w_p = jnp.pad(nb_weights.reshape(-1), (0, pad * t))

    hns = _make_sc_ns(in_f, bp, nw, nc)(h, ns_p)
    hq = _make_hq(h, Q_w, Q_b)
    agg = _make_sc_agg(in_f, hid, bp, t, nw, nc)(hq, nb_p, w_p)

    blk = 1000
    out = pl.pallas_call(
        functools.partial(_out_body, in_f=in_f),
        grid=(b // blk,),
        in_specs=[
            pl.BlockSpec((blk, in_f), lambda i: (i, 0)),
            pl.BlockSpec((blk, hid), lambda i: (i, 0)),
            pl.BlockSpec((blk, t), lambda i: (i, 0)),
            pl.BlockSpec((out_f, in_f + hid), lambda i: (0, 0)),
            pl.BlockSpec((1, out_f), lambda i: (0, 0)),
        ],
        out_specs=pl.BlockSpec((blk, out_f), lambda i: (i, 0)),
        out_shape=jax.ShapeDtypeStruct((b, out_f), jnp.float32),
    )(hns, agg, nb_weights, W_w, W_b.reshape(1, out_f))
    return out


# speedup vs baseline: 2.9681x; 1.2968x over previous
import functools

import jax
import jax.numpy as jnp
from jax import lax
from jax.experimental import pallas as pl
from jax.experimental.pallas import tpu as pltpu
from jax.experimental.pallas import tpu_sc as plsc


def _leaky(x):
    return jnp.where(x >= 0, x, 0.01 * x)


def _hq_body(h_ref, qw_ref, qb_ref, out_ref, *, hw):
    x = lax.dot_general(h_ref[...], qw_ref[...],
                        (((1,), (1,)), ((), ())),
                        preferred_element_type=jnp.float32)
    x = _leaky(x + qb_ref[...]).astype(jnp.bfloat16)
    xi = lax.bitcast_convert_type(x, jnp.uint16).astype(jnp.int32)
    out_ref[...] = xi[:, :hw] | (xi[:, hw:] << 16)


def _make_hq(h, Q_w, Q_b):
    n, in_f = h.shape
    hid = Q_w.shape[0]
    hw = hid // 2
    blk = 2000
    return pl.pallas_call(
        functools.partial(_hq_body, hw=hw),
        grid=(n // blk,),
        in_specs=[
            pl.BlockSpec((blk, in_f), lambda i: (i, 0)),
            pl.BlockSpec((hid, in_f), lambda i: (0, 0)),
            pl.BlockSpec((1, hid), lambda i: (0, 0)),
        ],
        out_specs=pl.BlockSpec((blk, hw), lambda i: (i, 0)),
        out_shape=jax.ShapeDtypeStruct((n, hw), jnp.int32),
    )(h, Q_w, Q_b.reshape(1, hid))


def _make_sc_ns(in_f, bp, nw, nc):
    b_per_w = bp // nw
    mesh = plsc.VectorSubcoreMesh(core_axis_name="c", subcore_axis_name="s")
    n_g = (b_per_w + 127) // 128

    @functools.partial(
        pl.kernel,
        mesh=mesh,
        out_type=jax.ShapeDtypeStruct((bp, in_f), jnp.float32),
        scratch_types=[
            pltpu.VMEM((b_per_w,), jnp.int32),
            pltpu.VMEM((b_per_w, in_f), jnp.float32),
            pltpu.SemaphoreType.DMA,
        ],
    )
    def sc_ns(h_hbm, ns_hbm, hns_out, nsidx_v, hrows_v, sem):
        wid = lax.axis_index("s") * nc + lax.axis_index("c")
        base = wid * b_per_w
        pltpu.sync_copy(ns_hbm.at[pl.ds(base, b_per_w)], nsidx_v)
        cps = []
        for k in range(n_g):
            sz = min(128, b_per_w - k * 128)
            cps.append(pltpu.async_copy(
                h_hbm.at[nsidx_v.at[pl.ds(k * 128, sz)]],
                hrows_v.at[pl.ds(k * 128, sz)], sem))
        for cp in cps:
            cp.wait()
        pltpu.sync_copy(hrows_v, hns_out.at[pl.ds(base, b_per_w)])

    return sc_ns


def _make_sc_agg(in_f, hid, bp, t, nw, nc):
    L = 16
    b_per_w = bp // nw
    C = 16
    G = (C * t) // 128
    n_j = hid // L
    nch = b_per_w // C

    mesh = plsc.VectorSubcoreMesh(core_axis_name="c", subcore_axis_name="s")

    @functools.partial(
        pl.kernel,
        mesh=mesh,
        compiler_params=pltpu.CompilerParams(use_tc_tiling_on_sc=False),
        out_type=jax.ShapeDtypeStruct((bp, hid), jnp.float32),
        scratch_types=[
            pltpu.VMEM((b_per_w * t,), jnp.int32),
            pltpu.VMEM((b_per_w * t,), jnp.float32),
            pltpu.VMEM((C * t, hid // 2), jnp.int32),
            pltpu.VMEM((C * t, hid // 2), jnp.int32),
            pltpu.VMEM((C, hid), jnp.float32),
            pltpu.VMEM((C, hid), jnp.float32),
            pltpu.SemaphoreType.DMA,
            pltpu.SemaphoreType.DMA,
            pltpu.SemaphoreType.DMA,
            pltpu.SemaphoreType.DMA,
        ],
    )
    def sc_agg(hq_hbm, nb_hbm, w_hbm, agg_out,
               idx_all, w_all, rows_a, rows_b, agg_a, agg_b,
               sem_a, sem_b, sem_oa, sem_ob):
        wid = lax.axis_index("s") * nc + lax.axis_index("c")
        base = wid * b_per_w

        pltpu.sync_copy(nb_hbm.at[pl.ds(base * t, b_per_w * t)], idx_all)
        pltpu.sync_copy(w_hbm.at[pl.ds(base * t, b_per_w * t)], w_all)

        def start_gathers(c, rows_v, sem):
            for k in range(G):
                pltpu.async_copy(
                    hq_hbm.at[idx_all.at[pl.ds(c * C * t + k * 128, 128)]],
                    rows_v.at[pl.ds(k * 128, 128)], sem)

        def wait_gathers(rows_v, sem):
            for k in range(G):
                pltpu.make_async_copy(
                    hq_hbm.at[idx_all.at[pl.ds(k * 128, 128)]],
                    rows_v.at[pl.ds(k * 128, 128)], sem).wait()

        def compute(c, rows_v, agg_v):
            def node(i, _):
                nb_off = (c * C + i) * t
                accs = [jnp.zeros((L,), jnp.float32) for _ in range(n_j)]
                wrow = [w_all[pl.ds(nb_off + g * L, L)]
                        for g in range(t // L)]
                for tt in range(t):
                    wspl = jnp.full((L,), wrow[tt // L][tt % L],
                                    dtype=jnp.float32)
                    for g in range(n_j // 2):
                        w32 = rows_v[i * t + tt, pl.ds(g * L, L)]
                        ua = lax.bitcast_convert_type(w32 << 16, jnp.float32)
                        ub = lax.bitcast_convert_type(
                            w32 & jnp.int32(-65536), jnp.float32)
                        accs[2 * g] = accs[2 * g] + wspl * ua
                        accs[2 * g + 1] = accs[2 * g + 1] + wspl * ub
                for j in range(n_j):
                    agg_v[i, pl.ds(j * L, L)] = accs[j]
                return 0

            lax.fori_loop(0, C, node, 0)

        def drain_out(agg_v, sem):
            pltpu.make_async_copy(agg_v, agg_out.at[pl.ds(0, C)], sem).wait()

        start_gathers(0, rows_a, sem_a)

        def pair(cp, _):
            c0 = 2 * cp
            c1 = c0 + 1
            start_gathers(c1, rows_b, sem_b)

            @pl.when(cp > 0)
            def _():
                drain_out(agg_a, sem_oa)
            wait_gathers(rows_a, sem_a)
            compute(c0, rows_a, agg_a)
            pltpu.async_copy(agg_a, agg_out.at[pl.ds(base + c0 * C, C)],
                             sem_oa)

            @pl.when(c1 + 1 < nch)
            def _():
                start_gathers(c1 + 1, rows_a, sem_a)

            @pl.when(cp > 0)
            def _():
                drain_out(agg_b, sem_ob)
            wait_gathers(rows_b, sem_b)
            compute(c1, rows_b, agg_b)
            pltpu.async_copy(agg_b, agg_out.at[pl.ds(base + c1 * C, C)],
                             sem_ob)
            return 0

        lax.fori_loop(0, nch // 2, pair, 0)
        drain_out(agg_a, sem_oa)
        drain_out(agg_b, sem_ob)

    return sc_agg


def _out_body(hns_ref, agg_ref, nbw_ref, w_ref, wb_ref, out_ref, *, in_f):
    ws = jnp.sum(nbw_ref[...], axis=1, keepdims=True)
    ws = jnp.where(ws == 0, 1.0, ws)
    agg = agg_ref[...] / ws
    w = w_ref[...]
    y = lax.dot_general(hns_ref[...], w[:, :in_f],
                        (((1,), (1,)), ((), ())),
                        preferred_element_type=jnp.float32)
    y = y + lax.dot_general(agg, w[:, in_f:],
                            (((1,), (1,)), ((), ())),
                            preferred_element_type=jnp.float32)
    y = _leaky(y + wb_ref[...])
    norm = jnp.sqrt(jnp.sum(y * y, axis=1, keepdims=True))
    norm = jnp.where(norm == 0, 1.0, norm)
    out_ref[...] = y / norm


def kernel(h, nodeset, nb_nodes, nb_weights, Q_w, Q_b, W_w, W_b):
    n_total, in_f = h.shape
    b, t = nb_nodes.shape
    hid = Q_w.shape[0]
    out_f = W_w.shape[0]

    info = plsc.get_sparse_core_info()
    nc, n_sub = info.num_cores, info.num_subcores
    nw = nc * n_sub

    chunk = 64 * nw
    bp = ((b + chunk - 1) // chunk) * chunk
    pad = bp - b
    ns_p = jnp.pad(nodeset.astype(jnp.int32), (0, pad))
    nb_p = jnp.pad(nb_nodes.astype(jnp.int32).reshape(-1), (0, pad * t))
    w_p = jnp.pad(nb_weights.reshape(-1), (0, pad * t))

    perm = []
    for g in range(hid // 32):
        perm.extend(range(32 * g, 32 * g + 16))
    for g in range(hid // 32):
        perm.extend(range(32 * g + 16, 32 * g + 32))
    perm = jnp.asarray(perm, dtype=jnp.int32)

    hns = _make_sc_ns(in_f, bp, nw, nc)(h, ns_p)
    hq = _make_hq(h, Q_w[perm, :], Q_b[perm])
    agg = _make_sc_agg(in_f, hid, bp, t, nw, nc)(hq, nb_p, w_p)

    blk = 1000
    out = pl.pallas_call(
        functools.partial(_out_body, in_f=in_f),
        grid=(b // blk,),
        in_specs=[
            pl.BlockSpec((blk, in_f), lambda i: (i, 0)),
            pl.BlockSpec((blk, hid), lambda i: (i, 0)),
            pl.BlockSpec((blk, t), lambda i: (i, 0)),
            pl.BlockSpec((out_f, in_f + hid), lambda i: (0, 0)),
            pl.BlockSpec((1, out_f), lambda i: (0, 0)),
        ],
        out_specs=pl.BlockSpec((blk, out_f), lambda i: (i, 0)),
        out_shape=jax.ShapeDtypeStruct((b, out_f), jnp.float32),
    )(hns, agg, nb_weights, W_w, W_b.reshape(1, out_f))
    return out


# speedup vs baseline: 2.9711x; 1.0010x over previous
import functools

import jax
import jax.numpy as jnp
from jax import lax
from jax.experimental import pallas as pl
from jax.experimental.pallas import tpu as pltpu
from jax.experimental.pallas import tpu_sc as plsc


def _leaky(x):
    return jnp.where(x >= 0, x, 0.01 * x)


def _hq_body(h_ref, qw_ref, qb_ref, out_ref, *, hw):
    x = lax.dot_general(h_ref[...], qw_ref[...],
                        (((1,), (1,)), ((), ())),
                        preferred_element_type=jnp.float32)
    x = _leaky(x + qb_ref[...]).astype(jnp.bfloat16)
    xi = lax.bitcast_convert_type(x, jnp.uint16).astype(jnp.int32)
    out_ref[...] = xi[:, :hw] | (xi[:, hw:] << 16)


def _make_hq(h, Q_w, Q_b):
    n, in_f = h.shape
    hid = Q_w.shape[0]
    hw = hid // 2
    blk = 2000
    return pl.pallas_call(
        functools.partial(_hq_body, hw=hw),
        grid=(n // blk,),
        in_specs=[
            pl.BlockSpec((blk, in_f), lambda i: (i, 0)),
            pl.BlockSpec((hid, in_f), lambda i: (0, 0)),
            pl.BlockSpec((1, hid), lambda i: (0, 0)),
        ],
        out_specs=pl.BlockSpec((blk, hw), lambda i: (i, 0)),
        out_shape=jax.ShapeDtypeStruct((n, hw), jnp.int32),
    )(h, Q_w, Q_b.reshape(1, hid))


def _make_sc_agg(in_f, hid, bp, t, nw, nc):
    L = 16
    b_per_w = bp // nw
    C = 16
    G = (C * t) // 128
    n_j = hid // L
    nch = b_per_w // C

    mesh = plsc.VectorSubcoreMesh(core_axis_name="c", subcore_axis_name="s")

    @functools.partial(
        pl.kernel,
        mesh=mesh,
        compiler_params=pltpu.CompilerParams(use_tc_tiling_on_sc=False),
        out_type=[
            jax.ShapeDtypeStruct((bp, hid), jnp.float32),
            jax.ShapeDtypeStruct((bp, in_f), jnp.float32),
        ],
        scratch_types=[
            pltpu.VMEM((b_per_w * t,), jnp.int32),
            pltpu.VMEM((b_per_w * t,), jnp.float32),
            pltpu.VMEM((C * t, hid // 2), jnp.int32),
            pltpu.VMEM((C * t, hid // 2), jnp.int32),
            pltpu.VMEM((C, hid), jnp.float32),
            pltpu.VMEM((C, hid), jnp.float32),
            pltpu.VMEM((b_per_w,), jnp.int32),
            pltpu.VMEM((b_per_w, in_f), jnp.float32),
            pltpu.SemaphoreType.DMA,
            pltpu.SemaphoreType.DMA,
            pltpu.SemaphoreType.DMA,
            pltpu.SemaphoreType.DMA,
            pltpu.SemaphoreType.DMA,
        ],
    )
    def sc_agg(hq_hbm, h_hbm, ns_hbm, nb_hbm, w_hbm, agg_out, hns_out,
               idx_all, w_all, rows_a, rows_b, agg_a, agg_b,
               nsidx_v, hrows_v, sem_a, sem_b, sem_oa, sem_ob, sem_ns):
        wid = lax.axis_index("s") * nc + lax.axis_index("c")
        base = wid * b_per_w

        pltpu.sync_copy(nb_hbm.at[pl.ds(base * t, b_per_w * t)], idx_all)
        pltpu.sync_copy(w_hbm.at[pl.ds(base * t, b_per_w * t)], w_all)

        pltpu.sync_copy(ns_hbm.at[pl.ds(base, b_per_w)], nsidx_v)
        ns_cps = []
        for k in range((b_per_w + 127) // 128):
            sz = min(128, b_per_w - k * 128)
            ns_cps.append(pltpu.async_copy(
                h_hbm.at[nsidx_v.at[pl.ds(k * 128, sz)]],
                hrows_v.at[pl.ds(k * 128, sz)], sem_ns))

        def start_gathers(c, rows_v, sem):
            for k in range(G):
                pltpu.async_copy(
                    hq_hbm.at[idx_all.at[pl.ds(c * C * t + k * 128, 128)]],
                    rows_v.at[pl.ds(k * 128, 128)], sem)

        def wait_gathers(rows_v, sem):
            for k in range(G):
                pltpu.make_async_copy(
                    hq_hbm.at[idx_all.at[pl.ds(k * 128, 128)]],
                    rows_v.at[pl.ds(k * 128, 128)], sem).wait()

        def compute(c, rows_v, agg_v):
            def node(i, _):
                nb_off = (c * C + i) * t
                accs = [jnp.zeros((L,), jnp.float32) for _ in range(n_j)]
                wrow = [w_all[pl.ds(nb_off + g * L, L)]
                        for g in range(t // L)]
                for tt in range(t):
                    wspl = jnp.full((L,), wrow[tt // L][tt % L],
                                    dtype=jnp.float32)
                    for g in range(n_j // 2):
                        w32 = rows_v[i * t + tt, pl.ds(g * L, L)]
                        ua = lax.bitcast_convert_type(w32 << 16, jnp.float32)
                        ub = lax.bitcast_convert_type(
                            w32 & jnp.int32(-65536), jnp.float32)
                        accs[2 * g] = accs[2 * g] + wspl * ua
                        accs[2 * g + 1] = accs[2 * g + 1] + wspl * ub
                for j in range(n_j):
                    agg_v[i, pl.ds(j * L, L)] = accs[j]
                return 0

            lax.fori_loop(0, C, node, 0)

        def drain_out(agg_v, sem):
            pltpu.make_async_copy(agg_v, agg_out.at[pl.ds(0, C)], sem).wait()

        start_gathers(0, rows_a, sem_a)

        def pair(cp, _):
            c0 = 2 * cp
            c1 = c0 + 1
            start_gathers(c1, rows_b, sem_b)

            @pl.when(cp > 0)
            def _():
                drain_out(agg_a, sem_oa)
            wait_gathers(rows_a, sem_a)
            compute(c0, rows_a, agg_a)
            pltpu.async_copy(agg_a, agg_out.at[pl.ds(base + c0 * C, C)],
                             sem_oa)

            @pl.when(c1 + 1 < nch)
            def _():
                start_gathers(c1 + 1, rows_a, sem_a)

            @pl.when(cp > 0)
            def _():
                drain_out(agg_b, sem_ob)
            wait_gathers(rows_b, sem_b)
            compute(c1, rows_b, agg_b)
            pltpu.async_copy(agg_b, agg_out.at[pl.ds(base + c1 * C, C)],
                             sem_ob)
            return 0

        lax.fori_loop(0, nch // 2, pair, 0)
        for cp in ns_cps:
            cp.wait()
        pltpu.sync_copy(hrows_v, hns_out.at[pl.ds(base, b_per_w)])
        drain_out(agg_a, sem_oa)
        drain_out(agg_b, sem_ob)

    return sc_agg


def _out_body(hns_ref, agg_ref, nbw_ref, w_ref, wb_ref, out_ref, *, in_f):
    ws = jnp.sum(nbw_ref[...], axis=1, keepdims=True)
    ws = jnp.where(ws == 0, 1.0, ws)
    agg = agg_ref[...] / ws
    w = w_ref[...]
    y = lax.dot_general(hns_ref[...], w[:, :in_f],
                        (((1,), (1,)), ((), ())),
                        preferred_element_type=jnp.float32)
    y = y + lax.dot_general(agg, w[:, in_f:],
                            (((1,), (1,)), ((), ())),
                            preferred_element_type=jnp.float32)
    y = _leaky(y + wb_ref[...])
    norm = jnp.sqrt(jnp.sum(y * y, axis=1, keepdims=True))
    norm = jnp.where(norm == 0, 1.0, norm)
    out_ref[...] = y / norm


def kernel(h, nodeset, nb_nodes, nb_weights, Q_w, Q_b, W_w, W_b):
    n_total, in_f = h.shape
    b, t = nb_nodes.shape
    hid = Q_w.shape[0]
    out_f = W_w.shape[0]

    info = plsc.get_sparse_core_info()
    nc, n_sub = info.num_cores, info.num_subcores
    nw = nc * n_sub

    chunk = 64 * nw
    bp = ((b + chunk - 1) // chunk) * chunk
    pad = bp - b
    ns_p = jnp.pad(nodeset.astype(jnp.int32), (0, pad))
    nb_p = jnp.pad(nb_nodes.astype(jnp.int32).reshape(-1), (0, pad * t))
    w_p = jnp.pad(nb_weights.reshape(-1), (0, pad * t))

    perm = []
    for g in range(hid // 32):
        perm.extend(range(32 * g, 32 * g + 16))
    for g in range(hid // 32):
        perm.extend(range(32 * g + 16, 32 * g + 32))
    perm = jnp.asarray(perm, dtype=jnp.int32)

    hq = _make_hq(h, Q_w[perm, :], Q_b[perm])
    hw = hid // 2
    hq = lax.optimization_barrier(hq.reshape(n_total // 4, 4 * hw))
    hq = hq.reshape(n_total, hw)
    agg, hns = _make_sc_agg(in_f, hid, bp, t, nw, nc)(
        hq, h, ns_p, nb_p, w_p)

    blk = 1000
    out = pl.pallas_call(
        functools.partial(_out_body, in_f=in_f),
        grid=(b // blk,),
        in_specs=[
            pl.BlockSpec((blk, in_f), lambda i: (i, 0)),
            pl.BlockSpec((blk, hid), lambda i: (i, 0)),
            pl.BlockSpec((blk, t), lambda i: (i, 0)),
            pl.BlockSpec((out_f, in_f + hid), lambda i: (0, 0)),
            pl.BlockSpec((1, out_f), lambda i: (0, 0)),
        ],
        out_specs=pl.BlockSpec((blk, out_f), lambda i: (i, 0)),
        out_shape=jax.ShapeDtypeStruct((b, out_f), jnp.float32),
    )(hns, agg, nb_weights, W_w, W_b.reshape(1, out_f))
    return out
